# Initial kernel scaffold; baseline (speedup 1.0000x reference)
import functools

import jax
import jax.numpy as jnp
from jax import lax
from jax.experimental import pallas as pl
from jax.experimental.pallas import tpu as pltpu
from jax.experimental.pallas import tpu_sc as plsc

N_ROWS = 65536
N_COLS = 65536
NNZ = 64
Q = 256
TOP_K = 10

NC = 2
NS = 16
L = 16
NW = NC * NS
QPT = Q // NS
HALF = N_ROWS // NC
SLICE = HALF // NS
NCHUNK = SLICE // L
DUMP = HALF
ACC = HALF + NS * L
INT_MAX = jnp.int32(2**31 - 1)


def _sc_accum_topk(qidx, qvals, rtab, vtab):
  mesh = plsc.VectorSubcoreMesh(
      core_axis_name="c", subcore_axis_name="s", num_cores=NC, num_subcores=NS)

  @functools.partial(
      pl.kernel,
      out_type=(
          jax.ShapeDtypeStruct((NW, L), jnp.float32),
          jax.ShapeDtypeStruct((NW, L), jnp.int32),
      ),
      mesh=mesh,
      scratch_types=[
          pltpu.VMEM((L,), jnp.int32),
          pltpu.VMEM((L,), jnp.float32),
          pltpu.VMEM((QPT, NNZ), jnp.int32),
          pltpu.VMEM((QPT, NNZ), jnp.float32),
          pltpu.VMEM((8, 128), jnp.int32),
          pltpu.VMEM((8, 128), jnp.float32),
          pltpu.VMEM((SLICE,), jnp.float32),
          pltpu.VMEM((L,), jnp.float32),
          pltpu.VMEM((L,), jnp.int32),
          pltpu.VMEM_SHARED((ACC,), jnp.float32),
          pltpu.SemaphoreType.DMA,
          pltpu.SemaphoreType.DMA,
      ],
  )
  def k(qidx_h, qval_h, rtab_h, vtab_h, ovals_h, oidx_h,
        qidx_v, qval_v, rrows, vrows, sidx, sval, slicebuf, cv, ci,
        acc, sem1, sem2):
    c = lax.axis_index("c")
    s = lax.axis_index("s")
    wid = c * NS + s
    qbase = s * QPT
    rbase = c * HALF
    iota = lax.iota(jnp.int32, L)

    pltpu.sync_copy(qidx_h.at[pl.ds(qbase, QPT)], qidx_v)
    pltpu.sync_copy(qval_h.at[pl.ds(qbase, QPT)], qval_v)
    cp1 = pltpu.async_copy(rtab_h.at[qidx_v], rrows, sem1)
    cp2 = pltpu.async_copy(vtab_h.at[qidx_v], vrows, sem2)

    zv = jnp.zeros((L,), jnp.float32)
    for i in range(NCHUNK):
      slicebuf[pl.ds(i * L, L)] = zv
    pltpu.sync_copy(slicebuf, acc.at[pl.ds(s * SLICE, SLICE)])

    cp1.wait()
    cp2.wait()

    dump = DUMP + s * L + iota
    for q in range(QPT):
      vq = plsc.load_gather(qval_v, [jnp.full((L,), q, jnp.int32)])
      for j in range(NNZ // L):
        r = rrows[q, pl.ds(j * L, L)]
        v = vrows[q, pl.ds(j * L, L)] * vq
        lr = r - rbase
        inb = (lr >= 0) & (lr < HALF)
        flat = q * NNZ + j * L
        sidx[flat // 128, pl.ds(flat % 128, L)] = jnp.where(inb, lr, dump)
        sval[flat // 128, pl.ds(flat % 128, L)] = v

    plsc.subcore_barrier()
    for jj in range(8):
      pltpu.sync_copy(sval.at[jj], acc.at[sidx.at[jj]], add=True)
    plsc.subcore_barrier()

    pltpu.sync_copy(acc.at[pl.ds(s * SLICE, SLICE)], slicebuf)
    gbase = rbase + s * SLICE
    tv = slicebuf[pl.ds(0, L)]
    ti = gbase + iota
    tv, ti = plsc.sort_key_val(tv, ti)

    def body(j, carry):
      tv, ti = carry
      cvv = slicebuf[pl.ds(j * L, L)]
      cii = gbase + j * L + iota
      cvv, cii = plsc.sort_key_val(cvv, cii, descending=True)
      take_t = tv >= cvv
      mv = jnp.where(take_t, tv, cvv)
      mi = jnp.where(take_t, ti, cii)
      return plsc.sort_key_val(mv, mi)

    tv, ti = lax.fori_loop(1, NCHUNK, body, (tv, ti))
    tv, ti = plsc.sort_key_val(tv, ti, descending=True)
    cv[...] = tv
    ci[...] = ti
    pltpu.sync_copy(cv, ovals_h.at[wid])
    pltpu.sync_copy(ci, oidx_h.at[wid])

  return k(qidx, qvals, rtab, vtab)


def _tc_merge(cand_v, cand_i):
  def body(cv_ref, ci_ref, ov_ref, oi_ref):
    v = cv_ref[...]
    ix = ci_ref[...]
    io = lax.broadcasted_iota(jnp.int32, (1, L), 1)
    ov = jnp.zeros((1, L), jnp.float32)
    oi = jnp.zeros((1, L), jnp.int32)
    for k in range(TOP_K):
      gmax = jnp.max(v)
      gidx = jnp.min(jnp.where(v == gmax, ix, INT_MAX))
      ov = jnp.where(io == k, gmax, ov)
      oi = jnp.where(io == k, gidx, oi)
      v = jnp.where((v == gmax) & (ix == gidx), -jnp.inf, v)
    ov_ref[...] = ov
    oi_ref[...] = oi

  return pl.pallas_call(
      body,
      out_shape=(
          jax.ShapeDtypeStruct((1, L), jnp.float32),
          jax.ShapeDtypeStruct((1, L), jnp.int32),
      ),
  )(cand_v, cand_i)


def kernel(indices, values, ccol, rindices, cvalues):
  del ccol
  qidx = indices.reshape(Q)
  qvals = values.reshape(Q)
  rtab = rindices.reshape(N_COLS, NNZ)
  vtab = cvalues.reshape(N_COLS, NNZ)
  cand_v, cand_i = _sc_accum_topk(qidx, qvals, rtab, vtab)
  ov, oi = _tc_merge(cand_v, cand_i)
  return ov[0, :TOP_K], oi[0, :TOP_K]


# speedup vs baseline: 4.2915x; 4.2915x over previous
import functools

import jax
import jax.numpy as jnp
from jax import lax
from jax.experimental import pallas as pl
from jax.experimental.pallas import tpu as pltpu
from jax.experimental.pallas import tpu_sc as plsc

N_ROWS = 65536
N_COLS = 65536
NNZ = 64
Q = 256
TOP_K = 10

NC = 2
NS = 16
L = 16
NW = NC * NS
QPT = Q // NS
HALF = N_ROWS // NC
SLICE = HALF // NS
NCHUNK = SLICE // L
DUMP = HALF
ACC = HALF + NS * L
INT_MAX = 2**31 - 1


def _sc_accum_topk(qidx, qvals, rtab, vtab):
  mesh = plsc.VectorSubcoreMesh(
      core_axis_name="c", subcore_axis_name="s", num_cores=NC, num_subcores=NS)

  @functools.partial(
      pl.kernel,
      out_type=(
          jax.ShapeDtypeStruct((NW, L), jnp.float32),
          jax.ShapeDtypeStruct((NW, L), jnp.int32),
      ),
      mesh=mesh,
      compiler_params=pltpu.CompilerParams(
          needs_layout_passes=False, use_tc_tiling_on_sc=False),
      scratch_types=[
          pltpu.VMEM((L,), jnp.int32),
          pltpu.VMEM((8, 128), jnp.float32),
          pltpu.VMEM((QPT, NNZ), jnp.int32),
          pltpu.VMEM((QPT, NNZ), jnp.float32),
          pltpu.VMEM((8, 128), jnp.int32),
          pltpu.VMEM((8, 128), jnp.float32),
          pltpu.VMEM((SLICE,), jnp.float32),
          pltpu.VMEM((L,), jnp.float32),
          pltpu.VMEM((L,), jnp.int32),
          pltpu.VMEM_SHARED((ACC,), jnp.float32),
          pltpu.SemaphoreType.DMA,
          pltpu.SemaphoreType.DMA,
      ],
  )
  def k(qidx_h, qve_h, rtab_h, vtab_h, ovals_h, oidx_h,
        qidx_v, qvexp, rrows, vrows, sidx, sval, slicebuf, cv, ci,
        acc, sem1, sem2):
    c = lax.axis_index("c")
    s = lax.axis_index("s")
    wid = c * NS + s
    qbase = s * QPT
    rbase = c * HALF
    iota = lax.iota(jnp.int32, L)

    pltpu.sync_copy(qidx_h.at[pl.ds(qbase, QPT)], qidx_v)
    pltpu.sync_copy(qve_h.at[s], qvexp)
    cp1 = pltpu.async_copy(rtab_h.at[qidx_v], rrows, sem1)
    cp2 = pltpu.async_copy(vtab_h.at[qidx_v], vrows, sem2)

    zv = jnp.zeros((L,), jnp.float32)
    for i in range(NCHUNK):
      slicebuf[pl.ds(i * L, L)] = zv
    pltpu.sync_copy(slicebuf, acc.at[pl.ds(s * SLICE, SLICE)])

    cp1.wait()
    cp2.wait()

    dump = DUMP + s * L + iota
    for q in range(QPT):
      for j in range(NNZ // L):
        flat = q * NNZ + j * L
        r = rrows[q, pl.ds(j * L, L)]
        vq = qvexp[flat // 128, pl.ds(flat % 128, L)]
        v = vrows[q, pl.ds(j * L, L)] * vq
        lr = r - rbase
        inb = (lr >= 0) & (lr < HALF)
        sidx[flat // 128, pl.ds(flat % 128, L)] = jnp.where(inb, lr, dump)
        sval[flat // 128, pl.ds(flat % 128, L)] = v

    plsc.subcore_barrier()
    for jj in range(8):
      pltpu.sync_copy(sval.at[jj], acc.at[sidx.at[jj]], add=True)
    plsc.subcore_barrier()

    pltpu.sync_copy(acc.at[pl.ds(s * SLICE, SLICE)], slicebuf)
    gbase = rbase + s * SLICE
    tv = slicebuf[pl.ds(0, L)]
    ti = gbase + iota
    tv, ti = plsc.sort_key_val(tv, ti)

    def body(j, carry):
      tv, ti = carry
      cvv = slicebuf[pl.ds(j * L, L)]
      cii = gbase + j * L + iota
      cvv, cii = plsc.sort_key_val(cvv, cii, descending=True)
      take_t = tv >= cvv
      mv = jnp.where(take_t, tv, cvv)
      mi = jnp.where(take_t, ti, cii)
      return tuple(plsc.sort_key_val(mv, mi))

    tv, ti = lax.fori_loop(1, NCHUNK, body, (tv, ti))
    tv, ti = plsc.sort_key_val(tv, ti, descending=True)
    cv[...] = tv
    ci[...] = ti
    pltpu.sync_copy(cv, ovals_h.at[wid])
    pltpu.sync_copy(ci, oidx_h.at[wid])

  return k(qidx, qvals, rtab, vtab)


def _tc_merge(cand_v, cand_i):
  def body(cv_ref, ci_ref, ov_ref, oi_ref):
    v = cv_ref[...]
    ix = ci_ref[...]
    io = lax.broadcasted_iota(jnp.int32, (1, L), 1)
    ov = jnp.zeros((1, L), jnp.float32)
    oi = jnp.zeros((1, L), jnp.int32)
    for k in range(TOP_K):
      gmax = jnp.max(v)
      gidx = jnp.min(jnp.where(v == gmax, ix, INT_MAX))
      ov = jnp.where(io == k, gmax, ov)
      oi = jnp.where(io == k, gidx, oi)
      v = jnp.where((v == gmax) & (ix == gidx), -jnp.inf, v)
    ov_ref[...] = ov
    oi_ref[...] = oi

  return pl.pallas_call(
      body,
      out_shape=(
          jax.ShapeDtypeStruct((1, L), jnp.float32),
          jax.ShapeDtypeStruct((1, L), jnp.int32),
      ),
  )(cand_v, cand_i)


def kernel(indices, values, ccol, rindices, cvalues):
  del ccol
  qidx = indices.reshape(Q)
  qve = jnp.repeat(values.reshape(Q), NNZ).reshape(NS, 8, 128)
  rtab = rindices.reshape(N_COLS, NNZ)
  vtab = cvalues.reshape(N_COLS, NNZ)
  cand_v, cand_i = _sc_accum_topk(qidx, qve, rtab, vtab)
  ov, oi = _tc_merge(cand_v, cand_i)
  return [ov[0, :TOP_K], oi[0, :TOP_K]]


# speedup vs baseline: 4.3067x; 1.0035x over previous
import functools

import jax
import jax.numpy as jnp
from jax import lax
from jax.experimental import pallas as pl
from jax.experimental.pallas import tpu as pltpu
from jax.experimental.pallas import tpu_sc as plsc

N_ROWS = 65536
N_COLS = 65536
NNZ = 64
Q = 256
TOP_K = 10

NC = 2
NS = 16
L = 16
NW = NC * NS
QPT = Q // NS
HALF = N_ROWS // NC
SLICE = HALF // NS
NCHUNK = SLICE // L
DUMP = HALF
ACC = HALF + NS * L
INT_MAX = 2**31 - 1


def _sc_accum_topk(qidx, qvals, rtab, vtab):
  mesh = plsc.VectorSubcoreMesh(
      core_axis_name="c", subcore_axis_name="s", num_cores=NC, num_subcores=NS)

  @functools.partial(
      pl.kernel,
      out_type=(
          jax.ShapeDtypeStruct((NW, L), jnp.float32),
          jax.ShapeDtypeStruct((NW, L), jnp.int32),
      ),
      mesh=mesh,
      compiler_params=pltpu.CompilerParams(
          needs_layout_passes=False, use_tc_tiling_on_sc=False),
      scratch_types=[
          pltpu.VMEM((L,), jnp.int32),
          pltpu.VMEM((L,), jnp.float32),
          pltpu.VMEM((QPT, NNZ), jnp.int32),
          pltpu.VMEM((QPT, NNZ), jnp.float32),
          pltpu.VMEM((8, 128), jnp.int32),
          pltpu.VMEM((8, 128), jnp.float32),
          pltpu.VMEM((SLICE,), jnp.float32),
          pltpu.VMEM((L,), jnp.float32),
          pltpu.VMEM((L,), jnp.int32),
          pltpu.VMEM_SHARED((ACC,), jnp.float32),
          pltpu.SemaphoreType.DMA,
          pltpu.SemaphoreType.DMA,
      ],
  )
  def k(qidx_h, qval_h, rtab_h, vtab_h, ovals_h, oidx_h,
        qidx_v, qval_v, rrows, vrows, sidx, sval, slicebuf, cv, ci,
        acc, sem1, sem2):
    c = lax.axis_index("c")
    s = lax.axis_index("s")
    wid = c * NS + s
    qbase = s * QPT
    rbase = c * HALF
    iota = lax.iota(jnp.int32, L)

    pltpu.sync_copy(qidx_h.at[pl.ds(qbase, QPT)], qidx_v)
    pltpu.sync_copy(qval_h.at[pl.ds(qbase, QPT)], qval_v)
    cp1 = pltpu.async_copy(rtab_h.at[qidx_v], rrows, sem1)
    cp2 = pltpu.async_copy(vtab_h.at[qidx_v], vrows, sem2)

    zv = jnp.zeros((L,), jnp.float32)
    for i in range(NCHUNK):
      slicebuf[pl.ds(i * L, L)] = zv
    pltpu.sync_copy(slicebuf, acc.at[pl.ds(s * SLICE, SLICE)])

    cp1.wait()
    cp2.wait()

    dump = DUMP + s * L + iota
    for q in range(QPT):
      vq = plsc.load_gather(qval_v, [jnp.full((L,), q, jnp.int32)])
      for j in range(NNZ // L):
        flat = q * NNZ + j * L
        r = rrows[q, pl.ds(j * L, L)]
        v = vrows[q, pl.ds(j * L, L)] * vq
        lr = r - rbase
        inb = (lr >= 0) & (lr < HALF)
        sidx[flat // 128, pl.ds(flat % 128, L)] = jnp.where(inb, lr, dump)
        sval[flat // 128, pl.ds(flat % 128, L)] = v

    plsc.subcore_barrier()
    for jj in range(8):
      pltpu.sync_copy(sval.at[jj], acc.at[sidx.at[jj]], add=True)
    plsc.subcore_barrier()

    pltpu.sync_copy(acc.at[pl.ds(s * SLICE, SLICE)], slicebuf)
    gbase = rbase + s * SLICE
    tv = slicebuf[pl.ds(0, L)]
    ti = gbase + iota
    tv, ti = plsc.sort_key_val(tv, ti)

    def body(j, carry):
      tv, ti = carry
      cvv = slicebuf[pl.ds(j * L, L)]
      cii = gbase + j * L + iota
      cvv, cii = plsc.sort_key_val(cvv, cii, descending=True)
      take_t = tv >= cvv
      mv = jnp.where(take_t, tv, cvv)
      mi = jnp.where(take_t, ti, cii)
      return tuple(plsc.sort_key_val(mv, mi))

    tv, ti = lax.fori_loop(1, NCHUNK, body, (tv, ti))
    tv, ti = plsc.sort_key_val(tv, ti, descending=True)
    cv[...] = tv
    ci[...] = ti
    pltpu.sync_copy(cv, ovals_h.at[wid])
    pltpu.sync_copy(ci, oidx_h.at[wid])

  return k(qidx, qvals, rtab, vtab)


def _tc_merge(cand_v, cand_i):
  def body(cv_ref, ci_ref, ov_ref, oi_ref):
    v = cv_ref[...]
    ix = ci_ref[...]
    io = lax.broadcasted_iota(jnp.int32, (1, L), 1)
    ov = jnp.zeros((1, L), jnp.float32)
    oi = jnp.zeros((1, L), jnp.int32)
    for k in range(TOP_K):
      gmax = jnp.max(v)
      gidx = jnp.min(jnp.where(v == gmax, ix, INT_MAX))
      ov = jnp.where(io == k, gmax, ov)
      oi = jnp.where(io == k, gidx, oi)
      v = jnp.where((v == gmax) & (ix == gidx), -jnp.inf, v)
    ov_ref[...] = ov
    oi_ref[...] = oi

  return pl.pallas_call(
      body,
      out_shape=(
          jax.ShapeDtypeStruct((1, L), jnp.float32),
          jax.ShapeDtypeStruct((1, L), jnp.int32),
      ),
  )(cand_v, cand_i)


def kernel(indices, values, ccol, rindices, cvalues):
  del ccol
  qidx = indices.reshape(Q)
  qvals = values.reshape(Q)
  rtab = rindices.reshape(N_COLS, NNZ)
  vtab = cvalues.reshape(N_COLS, NNZ)
  cand_v, cand_i = _sc_accum_topk(qidx, qvals, rtab, vtab)
  ov, oi = _tc_merge(cand_v, cand_i)
  return [ov[0, :TOP_K], oi[0, :TOP_K]]


# speedup vs baseline: 5.1833x; 1.2036x over previous
import functools

import jax
import jax.numpy as jnp
from jax import lax
from jax.experimental import pallas as pl
from jax.experimental.pallas import tpu as pltpu
from jax.experimental.pallas import tpu_sc as plsc

N_ROWS = 65536
N_COLS = 65536
NNZ = 64
Q = 256
TOP_K = 10

NS = 16
L = 16
QPT = Q // NS
HALF = N_ROWS // 2
SLICE = N_ROWS // NS
NCHUNK = SLICE // L


def _sc_retrieve(qidx, qvals, rtab, vtab):
  mesh = plsc.VectorSubcoreMesh(
      core_axis_name="c", subcore_axis_name="s", num_cores=2, num_subcores=NS)

  @functools.partial(
      pl.kernel,
      out_type=(
          jax.ShapeDtypeStruct((L,), jnp.float32),
          jax.ShapeDtypeStruct((L,), jnp.int32),
      ),
      mesh=mesh,
      compiler_params=pltpu.CompilerParams(
          needs_layout_passes=False, use_tc_tiling_on_sc=False),
      scratch_types=[
          pltpu.VMEM((L,), jnp.int32),
          pltpu.VMEM((8, 128), jnp.float32),
          pltpu.VMEM((QPT, NNZ), jnp.int32),
          pltpu.VMEM((QPT, NNZ), jnp.float32),
          pltpu.VMEM((8, 128), jnp.int32),
          pltpu.VMEM((8, 128), jnp.float32),
          pltpu.VMEM((8, 128), jnp.int32),
          pltpu.VMEM((8, 128), jnp.float32),
          pltpu.VMEM((SLICE,), jnp.float32),
          pltpu.VMEM((NS * L,), jnp.float32),
          pltpu.VMEM((NS * L,), jnp.int32),
          pltpu.VMEM_SHARED((HALF,), jnp.float32),
          pltpu.VMEM_SHARED((HALF,), jnp.float32),
          pltpu.VMEM_SHARED((NS * L,), jnp.float32),
          pltpu.VMEM_SHARED((NS * L,), jnp.int32),
          pltpu.SemaphoreType.DMA,
          pltpu.SemaphoreType.DMA,
      ],
  )
  def k(qidx_h, qve_h, rtab_h, vtab_h, ovals_h, oidx_h,
        qidx_v, qvexp, rrows, vrows, silo, svlo, sihi, svhi,
        slicebuf, mv, mi, acc_lo, acc_hi, candv, candi, sem1, sem2):
    s = lax.axis_index("s")
    c = lax.axis_index("c")
    qbase = s * QPT
    iota = lax.iota(jnp.int32, L)

    @pl.when(c == 0)
    def _work():
      pltpu.sync_copy(qidx_h.at[pl.ds(qbase, QPT)], qidx_v)
      pltpu.sync_copy(qve_h.at[s], qvexp)
      cp1 = pltpu.async_copy(rtab_h.at[qidx_v], rrows, sem1)
      cp2 = pltpu.async_copy(vtab_h.at[qidx_v], vrows, sem2)

      zv = jnp.zeros((L,), jnp.float32)
      for i in range(NCHUNK // 2):
        slicebuf[pl.ds(i * L, L)] = zv
      zslice = slicebuf.at[pl.ds(0, SLICE // 2)]
      pltpu.sync_copy(zslice, acc_lo.at[pl.ds(s * (SLICE // 2), SLICE // 2)])
      pltpu.sync_copy(zslice, acc_hi.at[pl.ds(s * (SLICE // 2), SLICE // 2)])

      cp1.wait()
      cp2.wait()

      spread = s * L + iota
      for q in range(QPT):
        for j in range(NNZ // L):
          flat = q * NNZ + j * L
          r = rrows[q, pl.ds(j * L, L)]
          v = (vrows[q, pl.ds(j * L, L)]
               * qvexp[flat // 128, pl.ds(flat % 128, L)])
          in_lo = r < HALF
          row, col = flat // 128, flat % 128
          silo[row, pl.ds(col, L)] = jnp.where(in_lo, r, spread)
          svlo[row, pl.ds(col, L)] = jnp.where(in_lo, v, 0.0)
          sihi[row, pl.ds(col, L)] = jnp.where(in_lo, spread, r - HALF)
          svhi[row, pl.ds(col, L)] = jnp.where(in_lo, 0.0, v)

      plsc.subcore_barrier()
      for jj in range(8):
        pltpu.sync_copy(svlo.at[jj], acc_lo.at[silo.at[jj]], add=True)
        pltpu.sync_copy(svhi.at[jj], acc_hi.at[sihi.at[jj]], add=True)
      plsc.subcore_barrier()

      @pl.when(s < NS // 2)
      def _read_lo():
        pltpu.sync_copy(acc_lo.at[pl.ds(s * SLICE, SLICE)], slicebuf)

      @pl.when(s >= NS // 2)
      def _read_hi():
        pltpu.sync_copy(
            acc_hi.at[pl.ds((s - NS // 2) * SLICE, SLICE)], slicebuf)

      gbase = s * SLICE
      tv = slicebuf[pl.ds(0, L)]
      ti = gbase + iota
      tv, ti = plsc.sort_key_val(tv, ti)

      def body(j, carry):
        tv, ti = carry
        cvv = slicebuf[pl.ds(j * L, L)]
        cii = gbase + j * L + iota
        cvv, cii = plsc.sort_key_val(cvv, cii, descending=True)
        take_t = tv >= cvv
        return tuple(plsc.sort_key_val(
            jnp.where(take_t, tv, cvv), jnp.where(take_t, ti, cii)))

      tv, ti = lax.fori_loop(1, NCHUNK, body, (tv, ti))
      tv, ti = plsc.sort_key_val(tv, ti, descending=True)

      mv[pl.ds(0, L)] = tv
      mi[pl.ds(0, L)] = ti
      pltpu.sync_copy(mv.at[pl.ds(0, L)], candv.at[pl.ds(s * L, L)])
      pltpu.sync_copy(mi.at[pl.ds(0, L)], candi.at[pl.ds(s * L, L)])
      plsc.subcore_barrier()

      @pl.when(s == 0)
      def _merge():
        pltpu.sync_copy(candv, mv)
        pltpu.sync_copy(candi, mi)
        fv = mv[pl.ds(0, L)]
        fi = mi[pl.ds(0, L)]
        fv, fi = plsc.sort_key_val(fv, fi)

        def mbody(t, carry):
          fv, fi = carry
          cvv = mv[pl.ds(t * L, L)]
          cii = mi[pl.ds(t * L, L)]
          take_t = fv >= cvv
          return tuple(plsc.sort_key_val(
              jnp.where(take_t, fv, cvv), jnp.where(take_t, fi, cii)))

        fv, fi = lax.fori_loop(1, NS, mbody, (fv, fi))
        fv, fi = plsc.sort_key_val(fv, fi, descending=True)
        mv[pl.ds(0, L)] = fv
        mi[pl.ds(0, L)] = fi
        pltpu.sync_copy(mv.at[pl.ds(0, L)], ovals_h)
        pltpu.sync_copy(mi.at[pl.ds(0, L)], oidx_h)

  return k(qidx, qvals, rtab, vtab)


def kernel(indices, values, ccol, rindices, cvalues):
  del ccol
  qidx = indices.reshape(Q)
  qve = jnp.repeat(values.reshape(Q), NNZ).reshape(NS, 8, 128)
  rtab = rindices.reshape(N_COLS, NNZ)
  vtab = cvalues.reshape(N_COLS, NNZ)
  ov, oi = _sc_retrieve(qidx, qve, rtab, vtab)
  return [ov[:TOP_K], oi[:TOP_K]]
